# Initial kernel scaffold; baseline (speedup 1.0000x reference)
import jax
import jax.numpy as jnp
from jax.experimental import pallas as pl


def kernel(X, table):
    raise NotImplementedError("write your pallas kernel here")


# speedup vs baseline: 6.1256x; 6.1256x over previous
import functools

import jax
import jax.numpy as jnp
from jax import lax
from jax.experimental import pallas as pl
from jax.experimental.pallas import tpu as pltpu
from jax.experimental.pallas import tpu_sc as plsc

_NC = 2
_NS = 16
_NW = _NC * _NS

_ROW = 128
_K = 8


def _gather_call(R, V, D):
    rows_per_w = R // _NW
    n_iter = rows_per_w // _K
    mesh = plsc.VectorSubcoreMesh(core_axis_name="c", subcore_axis_name="s")

    @functools.partial(
        pl.kernel,
        mesh=mesh,
        out_type=jax.ShapeDtypeStruct((R, _ROW, D), jnp.float32),
        scratch_types=[
            pltpu.VMEM((_K, _ROW), jnp.int32),
            pltpu.VMEM((_K, _ROW, D), jnp.float32),
            pltpu.SemaphoreType.DMA,
        ],
        compiler_params=pltpu.CompilerParams(use_tc_tiling_on_sc=False),
    )
    def run(table_hbm, idx_hbm, out_hbm, idx_v, rows_v, sem):
        wid = lax.axis_index("s") * _NC + lax.axis_index("c")
        base = wid * rows_per_w

        def body(it, carry):
            rb = base + it * _K
            pltpu.sync_copy(idx_hbm.at[pl.ds(rb, _K)], idx_v)
            copies = [
                pltpu.make_async_copy(
                    table_hbm.at[idx_v.at[j]], rows_v.at[j], sem
                )
                for j in range(_K)
            ]
            for c in copies:
                c.start()
            for c in copies:
                c.wait()
            pltpu.sync_copy(rows_v, out_hbm.at[pl.ds(rb, _K)])
            return carry

        lax.fori_loop(0, n_iter, body, 0)

    return run


def kernel(X, table):
    B, H = X.shape
    V, D = table.shape
    R = (B * H) // _ROW
    idx = X.reshape(R, _ROW)
    out = _gather_call(R, V, D)(table, idx)
    return out.reshape(B, H, D)


# speedup vs baseline: 6.4991x; 1.0610x over previous
import functools

import jax
import jax.numpy as jnp
from jax import lax
from jax.experimental import pallas as pl
from jax.experimental.pallas import tpu as pltpu
from jax.experimental.pallas import tpu_sc as plsc

_NC = 2
_NS = 16
_NW = _NC * _NS

_ROW = 128
_K = 8


def _gather_call(R, V, D):
    rows_per_w = R // _NW
    n = rows_per_w // _K
    mesh = plsc.VectorSubcoreMesh(core_axis_name="c", subcore_axis_name="s")

    @functools.partial(
        pl.kernel,
        mesh=mesh,
        out_type=jax.ShapeDtypeStruct((R, _ROW, D), jnp.float32),
        scratch_types=[
            pltpu.VMEM((_K, _ROW), jnp.int32),
            pltpu.VMEM((_K, _ROW), jnp.int32),
            pltpu.VMEM((_K, _ROW, D), jnp.float32),
            pltpu.VMEM((_K, _ROW, D), jnp.float32),
            pltpu.SemaphoreType.DMA,
            pltpu.SemaphoreType.DMA,
            pltpu.SemaphoreType.DMA,
            pltpu.SemaphoreType.DMA,
            pltpu.SemaphoreType.DMA,
            pltpu.SemaphoreType.DMA,
        ],
        compiler_params=pltpu.CompilerParams(use_tc_tiling_on_sc=False),
    )
    def run(table_hbm, idx_hbm, out_hbm, idx0, idx1, rows0, rows1,
            isem0, isem1, gsem0, gsem1, ssem0, ssem1):
        wid = lax.axis_index("s") * _NC + lax.axis_index("c")
        base = wid * rows_per_w

        def idx_cp(g, idx_b, isem_b):
            return pltpu.make_async_copy(
                idx_hbm.at[pl.ds(base + g * _K, _K)], idx_b, isem_b)

        def gath_cp(idx_b, rows_b, gsem_b, j):
            return pltpu.make_async_copy(
                table_hbm.at[idx_b.at[j]], rows_b.at[j], gsem_b)

        def store_cp(g, rows_b, ssem_b):
            return pltpu.make_async_copy(
                rows_b, out_hbm.at[pl.ds(base + g * _K, _K)], ssem_b)

        def fire_gathers(idx_b, rows_b, gsem_b):
            for j in range(_K):
                gath_cp(idx_b, rows_b, gsem_b, j).start()

        def wait_gathers(idx_b, rows_b, gsem_b):
            for j in range(_K):
                gath_cp(idx_b, rows_b, gsem_b, j).wait()

        idx_cp(0, idx0, isem0).start()
        idx_cp(0, idx0, isem0).wait()
        fire_gathers(idx0, rows0, gsem0)
        idx_cp(1, idx1, isem1).start()
        idx_cp(1, idx1, isem1).wait()
        fire_gathers(idx1, rows1, gsem1)
        wait_gathers(idx0, rows0, gsem0)
        store_cp(0, rows0, ssem0).start()
        idx_cp(2, idx0, isem0).start()

        def body(t, carry):
            g0 = 2 * t
            idx_cp(g0, idx0, isem0).wait()
            store_cp(g0 - 2, rows0, ssem0).wait()
            fire_gathers(idx0, rows0, gsem0)
            wait_gathers(idx1, rows1, gsem1)
            store_cp(g0 - 1, rows1, ssem1).start()
            idx_cp(g0 + 1, idx1, isem1).start()
            idx_cp(g0 + 1, idx1, isem1).wait()
            store_cp(g0 - 1, rows1, ssem1).wait()
            fire_gathers(idx1, rows1, gsem1)
            wait_gathers(idx0, rows0, gsem0)
            store_cp(g0, rows0, ssem0).start()

            @pl.when(g0 + 2 < n)
            def _():
                idx_cp(g0 + 2, idx0, isem0).start()

            return carry

        lax.fori_loop(1, n // 2, body, 0)

        wait_gathers(idx1, rows1, gsem1)
        store_cp(n - 1, rows1, ssem1).start()
        store_cp(n - 2, rows0, ssem0).wait()
        store_cp(n - 1, rows1, ssem1).wait()

    return run


def kernel(X, table):
    B, H = X.shape
    V, D = table.shape
    R = (B * H) // _ROW
    idx = X.reshape(R, _ROW)
    out = _gather_call(R, V, D)(table, idx)
    return out.reshape(B, H, D)


# speedup vs baseline: 6.5166x; 1.0027x over previous
import functools

import jax
import jax.numpy as jnp
from jax import lax
from jax.experimental import pallas as pl
from jax.experimental.pallas import tpu as pltpu
from jax.experimental.pallas import tpu_sc as plsc

_NC = 2
_NS = 16
_NW = _NC * _NS

_CB = 4


def _gather_call(B, H, V, D):
    b_per_w = B // _NW
    n = b_per_w // _CB
    segs = []
    off = 0
    while off < H:
        sz = min(128, H - off)
        segs.append((off, sz))
        off += sz
    mesh = plsc.VectorSubcoreMesh(core_axis_name="c", subcore_axis_name="s")

    @functools.partial(
        pl.kernel,
        mesh=mesh,
        out_type=jax.ShapeDtypeStruct((B, H, D), jnp.float32),
        scratch_types=[
            pltpu.VMEM((_CB, H), jnp.int32),
            pltpu.VMEM((_CB, H), jnp.int32),
            pltpu.VMEM((_CB, H, D), jnp.float32),
            pltpu.VMEM((_CB, H, D), jnp.float32),
            pltpu.SemaphoreType.DMA,
            pltpu.SemaphoreType.DMA,
            pltpu.SemaphoreType.DMA,
            pltpu.SemaphoreType.DMA,
            pltpu.SemaphoreType.DMA,
            pltpu.SemaphoreType.DMA,
        ],
        compiler_params=pltpu.CompilerParams(use_tc_tiling_on_sc=False),
    )
    def run(table_hbm, idx_hbm, out_hbm, idx0, idx1, rows0, rows1,
            isem0, isem1, gsem0, gsem1, ssem0, ssem1):
        wid = lax.axis_index("s") * _NC + lax.axis_index("c")
        base = wid * b_per_w

        def idx_cp(g, idx_b, isem_b):
            return pltpu.make_async_copy(
                idx_hbm.at[pl.ds(base + g * _CB, _CB)], idx_b, isem_b)

        def gath_cp(idx_b, rows_b, gsem_b, i, off, sz):
            return pltpu.make_async_copy(
                table_hbm.at[idx_b.at[i, pl.ds(off, sz)]],
                rows_b.at[i, pl.ds(off, sz)],
                gsem_b)

        def store_cp(g, rows_b, ssem_b):
            return pltpu.make_async_copy(
                rows_b, out_hbm.at[pl.ds(base + g * _CB, _CB)], ssem_b)

        def fire_gathers(idx_b, rows_b, gsem_b):
            for i in range(_CB):
                for off, sz in segs:
                    gath_cp(idx_b, rows_b, gsem_b, i, off, sz).start()

        def wait_gathers(idx_b, rows_b, gsem_b):
            for i in range(_CB):
                for off, sz in segs:
                    gath_cp(idx_b, rows_b, gsem_b, i, off, sz).wait()

        idx_cp(0, idx0, isem0).start()
        idx_cp(0, idx0, isem0).wait()
        fire_gathers(idx0, rows0, gsem0)
        idx_cp(1, idx1, isem1).start()
        idx_cp(1, idx1, isem1).wait()
        fire_gathers(idx1, rows1, gsem1)
        wait_gathers(idx0, rows0, gsem0)
        store_cp(0, rows0, ssem0).start()
        idx_cp(2, idx0, isem0).start()

        def body(t, carry):
            g0 = 2 * t
            idx_cp(g0, idx0, isem0).wait()
            store_cp(g0 - 2, rows0, ssem0).wait()
            fire_gathers(idx0, rows0, gsem0)
            wait_gathers(idx1, rows1, gsem1)
            store_cp(g0 - 1, rows1, ssem1).start()
            idx_cp(g0 + 1, idx1, isem1).start()
            idx_cp(g0 + 1, idx1, isem1).wait()
            store_cp(g0 - 1, rows1, ssem1).wait()
            fire_gathers(idx1, rows1, gsem1)
            wait_gathers(idx0, rows0, gsem0)
            store_cp(g0, rows0, ssem0).start()

            @pl.when(g0 + 2 < n)
            def _():
                idx_cp(g0 + 2, idx0, isem0).start()

            return carry

        lax.fori_loop(1, n // 2, body, 0)

        wait_gathers(idx1, rows1, gsem1)
        store_cp(n - 1, rows1, ssem1).start()
        store_cp(n - 2, rows0, ssem0).wait()
        store_cp(n - 1, rows1, ssem1).wait()

    return run


def kernel(X, table):
    B, H = X.shape
    V, D = table.shape
    return _gather_call(B, H, V, D)(table, X)
